# baseline (speedup 1.0000x reference)
import functools

import jax
import jax.numpy as jnp
from jax import lax
from jax.experimental import pallas as pl
from jax.experimental.pallas import tpu as pltpu
from jax.experimental.pallas import tpu_sc as plsc

_NW = 32
_CHUNK = 64
_NCHUNK = 4


def _sc_gather_body(nch, x_hbm, table_hbm, out_hbm, idx_v, buf0, buf1, gsem,
                    wsem):
    wid = lax.axis_index("s") * 2 + lax.axis_index("c")
    pltpu.sync_copy(x_hbm.at[wid], idx_v)
    bufs = (buf0, buf1)
    base = wid * (nch * _CHUNK)

    gathers = [None, None]
    writes = [None, None]
    for j in range(nch):
        b = j % 2
        if j >= 2:
            writes[b].wait()
        gathers[b] = pltpu.async_copy(table_hbm.at[idx_v.at[j]], bufs[b], gsem)
        if j >= 1:
            pb = (j - 1) % 2
            gathers[pb].wait()
            writes[pb] = pltpu.async_copy(
                bufs[pb], out_hbm.at[pl.ds(base + (j - 1) * _CHUNK, _CHUNK)],
                wsem)
    lb = (nch - 1) % 2
    gathers[lb].wait()
    writes[lb] = pltpu.async_copy(
        bufs[lb], out_hbm.at[pl.ds(base + (nch - 1) * _CHUNK, _CHUNK)], wsem)
    if nch >= 2:
        writes[(nch - 2) % 2].wait()
    writes[lb].wait()


def _make_sc_gather(V, D, rows):
    nch = rows // (_NW * _CHUNK)
    mesh = plsc.VectorSubcoreMesh(core_axis_name="c", subcore_axis_name="s")
    return pl.kernel(
        functools.partial(_sc_gather_body, nch),
        out_type=jax.ShapeDtypeStruct((rows, D), jnp.float32),
        mesh=mesh,
        scratch_types=[
            pltpu.VMEM((nch, _CHUNK), jnp.int32),
            pltpu.VMEM((_CHUNK, D), jnp.float32),
            pltpu.VMEM((_CHUNK, D), jnp.float32),
            pltpu.SemaphoreType.DMA,
            pltpu.SemaphoreType.DMA,
        ],
    )


def _tc_partial_body(emb_ref, wt_ref, out_ref):
    z = jnp.dot(emb_ref[...].astype(jnp.bfloat16), wt_ref[...],
                preferred_element_type=jnp.float32)
    out_ref[...] = jnp.max(z, axis=0, keepdims=True)[None]


def _tc_final_body(parts_ref, bias_ref, out_ref):
    m = jnp.max(parts_ref[...], axis=1)
    out_ref[...] = jnp.tanh(m + bias_ref[...])


def kernel(x, bs, embedding_weight, W_c_weight, W_c_bias):
    B, L = x.shape
    V, D = embedding_weight.shape
    E = W_c_weight.shape[0]
    lc = L // _NCHUNK
    rows = B * lc

    xi = x.astype(jnp.int32)
    wt = W_c_weight.T.astype(jnp.bfloat16)
    bias = W_c_bias.reshape(1, E)
    sc_gather = _make_sc_gather(V, D, rows)

    parts = []
    for c in range(_NCHUNK):
        xc = xi[:, c * lc:(c + 1) * lc].reshape(_NW, -1, _CHUNK)
        emb = sc_gather(xc, embedding_weight)
        part = pl.pallas_call(
            _tc_partial_body,
            grid=(B,),
            in_specs=[
                pl.BlockSpec((lc, D), lambda b: (b, 0)),
                pl.BlockSpec((D, E), lambda b: (0, 0)),
            ],
            out_specs=pl.BlockSpec((1, 1, E), lambda b: (b, 0, 0)),
            out_shape=jax.ShapeDtypeStruct((B, 1, E), jnp.float32),
        )(emb, wt)
        parts.append(part)

    stacked = jnp.concatenate(parts, axis=1)
    out = pl.pallas_call(
        _tc_final_body,
        out_shape=jax.ShapeDtypeStruct((B, E), jnp.float32),
    )(stacked, bias)
    return out
